# baseline (speedup 1.0000x reference)
import functools

import jax
import jax.numpy as jnp
from jax import lax
from jax.experimental import pallas as pl
from jax.experimental.pallas import tpu as pltpu
from jax.experimental.pallas import tpu_sc as plsc

BATCH = 4096
DIM = 64
WIDE = 128
NUM_CORES = 2
NUM_SUBCORES = 16
NUM_WORKERS = NUM_CORES * NUM_SUBCORES
ROWS_PER_WORKER = BATCH // NUM_WORKERS
LANES = 16
ROWS_PER_BLOCK = 256


def _sc_gather(user_ids, item_ids, uemb, iemb, ubias128, ibias128):
    mesh = plsc.VectorSubcoreMesh(core_axis_name="c", subcore_axis_name="s")
    out_type = (
        jax.ShapeDtypeStruct((BATCH, DIM), jnp.float32),
        jax.ShapeDtypeStruct((BATCH, DIM), jnp.float32),
        jax.ShapeDtypeStruct((BATCH,), jnp.float32),
    )

    @functools.partial(
        pl.kernel,
        mesh=mesh,
        out_type=out_type,
        compiler_params=pltpu.CompilerParams(
            use_tc_tiling_on_sc=True, needs_layout_passes=False),
        scratch_types=[
            pltpu.VMEM((ROWS_PER_WORKER,), jnp.int32),
            pltpu.VMEM((ROWS_PER_WORKER,), jnp.int32),
            pltpu.VMEM((ROWS_PER_WORKER,), jnp.int32),
            pltpu.VMEM((ROWS_PER_WORKER,), jnp.int32),
            pltpu.VMEM((ROWS_PER_WORKER, WIDE), jnp.float32),
            pltpu.VMEM((ROWS_PER_WORKER, WIDE), jnp.float32),
            pltpu.VMEM((ROWS_PER_WORKER,), jnp.float32),
            pltpu.SemaphoreType.DMA,
            pltpu.SemaphoreType.DMA,
        ],
    )
    def k(uid_hbm, iid_hbm, uemb_hbm, iemb_hbm, ub_hbm, ib_hbm,
          urows_out, irows_out, b_out,
          uidx_v, iidx_v, ubrow_v, ibrow_v,
          ubrows_v, ibrows_v, bsel_v, sem, bsem):
        wid = lax.axis_index("s") * NUM_CORES + lax.axis_index("c")
        base = wid * ROWS_PER_WORKER
        pltpu.sync_copy(uid_hbm.at[pl.ds(base, ROWS_PER_WORKER)], uidx_v)
        pltpu.sync_copy(iid_hbm.at[pl.ds(base, ROWS_PER_WORKER)], iidx_v)

        @pl.loop(0, ROWS_PER_WORKER, step=LANES)
        def _(o):
            sl = pl.ds(o, LANES)
            ubrow_v[sl] = lax.shift_right_logical(uidx_v[sl], 7)
            ibrow_v[sl] = lax.shift_right_logical(iidx_v[sl], 7)

        cb1 = pltpu.async_copy(ub_hbm.at[ubrow_v], ubrows_v, bsem)
        cb2 = pltpu.async_copy(ib_hbm.at[ibrow_v], ibrows_v, bsem)

        @pl.loop(0, ROWS_PER_WORKER, step=LANES)
        def _(o):
            uv = uidx_v[pl.ds(o, LANES)]
            iv = iidx_v[pl.ds(o, LANES)]
            for jj in range(LANES):
                pltpu.async_copy(uemb_hbm.at[pl.ds(uv[jj], 1)],
                                 urows_out.at[pl.ds(base + o + jj, 1)], sem)
                pltpu.async_copy(iemb_hbm.at[pl.ds(iv[jj], 1)],
                                 irows_out.at[pl.ds(base + o + jj, 1)], sem)

        pltpu.make_async_copy(
            uemb_hbm.at[pl.ds(0, ROWS_PER_WORKER)],
            urows_out.at[pl.ds(base, ROWS_PER_WORKER)], sem).wait()
        pltpu.make_async_copy(
            iemb_hbm.at[pl.ds(0, ROWS_PER_WORKER)],
            irows_out.at[pl.ds(base, ROWS_PER_WORKER)], sem).wait()
        cb1.wait()
        cb2.wait()

        @pl.loop(0, ROWS_PER_WORKER, step=LANES)
        def _(o):
            sl = pl.ds(o, LANES)
            row_idx = o + lax.iota(jnp.int32, 16)
            ug = plsc.load_gather(
                ubrows_v, [row_idx, uidx_v[sl] & (WIDE - 1)])
            ig = plsc.load_gather(
                ibrows_v, [row_idx, iidx_v[sl] & (WIDE - 1)])
            bsel_v[sl] = ug + ig

        pltpu.sync_copy(bsel_v, b_out.at[pl.ds(base, ROWS_PER_WORKER)])

    return k(user_ids, item_ids, uemb, iemb, ubias128, ibias128)


def _tc_dot_body(u_ref, i_ref, dot_ref):
    dot_ref[...] = jnp.sum(u_ref[...] * i_ref[...], axis=1).reshape(1, BATCH)


def _tc_dot(urows, irows):
    return pl.pallas_call(
        _tc_dot_body,
        out_shape=jax.ShapeDtypeStruct((1, BATCH), jnp.float32),
    )(urows, irows)


def _tc_bcast_body(dot_ref, b_ref, out_ref):
    out_ref[...] = dot_ref[...] + b_ref[...]


def _tc_broadcast(dot_row, bias):
    grid = (BATCH // ROWS_PER_BLOCK,)
    return pl.pallas_call(
        _tc_bcast_body,
        grid=grid,
        in_specs=[
            pl.BlockSpec((1, BATCH), lambda i: (0, 0)),
            pl.BlockSpec((ROWS_PER_BLOCK, 1), lambda i: (i, 0)),
        ],
        out_specs=pl.BlockSpec((ROWS_PER_BLOCK, BATCH), lambda i: (i, 0)),
        out_shape=jax.ShapeDtypeStruct((BATCH, BATCH), jnp.float32),
    )(dot_row, bias)


def kernel(user_ids, item_ids, user_emb_table, item_emb_table,
           user_bias_table, item_bias_table):
    def pad128(t):
        n = t.shape[0]
        pad = (-n) % WIDE
        flat = t.reshape(-1)
        if pad:
            flat = jnp.pad(flat, (0, pad))
        return flat.reshape(-1, WIDE)

    ubias128 = pad128(user_bias_table)
    ibias128 = pad128(item_bias_table)
    urows, irows, bias = _sc_gather(
        user_ids, item_ids, user_emb_table, item_emb_table,
        ubias128, ibias128)
    dot_row = _tc_dot(urows, irows)
    return _tc_broadcast(dot_row, bias.reshape(BATCH, 1))


# speedup vs baseline: 1.9384x; 1.9384x over previous
import functools

import jax
import jax.numpy as jnp
from jax import lax
from jax.experimental import pallas as pl
from jax.experimental.pallas import tpu as pltpu
from jax.experimental.pallas import tpu_sc as plsc

BATCH = 4096
DIM = 64
WIDE = 128
NUM_CORES = 2
NUM_SUBCORES = 16
NUM_WORKERS = NUM_CORES * NUM_SUBCORES
ROWS_PER_WORKER = BATCH // NUM_WORKERS
LANES = 16
ROWS_PER_BLOCK = 512


def _sc_gather(user_ids, item_ids, uemb, iemb, ubias128, ibias128):
    mesh = plsc.VectorSubcoreMesh(core_axis_name="c", subcore_axis_name="s")
    out_type = (
        jax.ShapeDtypeStruct((BATCH, DIM), jnp.float32),
        jax.ShapeDtypeStruct((BATCH, DIM), jnp.float32),
        jax.ShapeDtypeStruct((BATCH,), jnp.float32),
    )

    @functools.partial(
        pl.kernel,
        mesh=mesh,
        out_type=out_type,
        compiler_params=pltpu.CompilerParams(
            use_tc_tiling_on_sc=True, needs_layout_passes=False),
        scratch_types=[
            pltpu.VMEM((ROWS_PER_WORKER,), jnp.int32),
            pltpu.VMEM((ROWS_PER_WORKER,), jnp.int32),
            pltpu.VMEM((ROWS_PER_WORKER,), jnp.int32),
            pltpu.VMEM((ROWS_PER_WORKER,), jnp.int32),
            pltpu.VMEM((ROWS_PER_WORKER, DIM), jnp.float32),
            pltpu.VMEM((ROWS_PER_WORKER, DIM), jnp.float32),
            pltpu.VMEM((ROWS_PER_WORKER, WIDE), jnp.float32),
            pltpu.VMEM((ROWS_PER_WORKER, WIDE), jnp.float32),
            pltpu.VMEM((ROWS_PER_WORKER,), jnp.float32),
            pltpu.SemaphoreType.DMA,
            pltpu.SemaphoreType.DMA,
        ],
    )
    def k(uid_hbm, iid_hbm, uemb_hbm, iemb_hbm, ub_hbm, ib_hbm,
          urows_out, irows_out, b_out,
          uidx_v, iidx_v, ubrow_v, ibrow_v,
          urows_v, irows_v, ubrows_v, ibrows_v, bsel_v, sem, bsem):
        wid = lax.axis_index("s") * NUM_CORES + lax.axis_index("c")
        base = wid * ROWS_PER_WORKER
        pltpu.sync_copy(uid_hbm.at[pl.ds(base, ROWS_PER_WORKER)], uidx_v)
        pltpu.sync_copy(iid_hbm.at[pl.ds(base, ROWS_PER_WORKER)], iidx_v)

        @pl.loop(0, ROWS_PER_WORKER, step=LANES)
        def _(o):
            sl = pl.ds(o, LANES)
            ubrow_v[sl] = lax.shift_right_logical(uidx_v[sl], 7)
            ibrow_v[sl] = lax.shift_right_logical(iidx_v[sl], 7)

        cb1 = pltpu.async_copy(ub_hbm.at[ubrow_v], ubrows_v, bsem)
        cb2 = pltpu.async_copy(ib_hbm.at[ibrow_v], ibrows_v, bsem)

        @pl.loop(0, ROWS_PER_WORKER, step=LANES)
        def _(o):
            uv = uidx_v[pl.ds(o, LANES)]
            iv = iidx_v[pl.ds(o, LANES)]
            for jj in range(LANES):
                pltpu.async_copy(uemb_hbm.at[pl.ds(uv[jj], 1)],
                                 urows_v.at[pl.ds(o + jj, 1)], sem)
                pltpu.async_copy(iemb_hbm.at[pl.ds(iv[jj], 1)],
                                 irows_v.at[pl.ds(o + jj, 1)], sem)

        pltpu.make_async_copy(
            uemb_hbm.at[pl.ds(0, ROWS_PER_WORKER)], urows_v, sem).wait()
        pltpu.make_async_copy(
            iemb_hbm.at[pl.ds(0, ROWS_PER_WORKER)], irows_v, sem).wait()
        cb1.wait()
        cb2.wait()

        @pl.loop(0, ROWS_PER_WORKER, step=LANES)
        def _(o):
            sl = pl.ds(o, LANES)
            row_idx = o + lax.iota(jnp.int32, 16)
            ug = plsc.load_gather(
                ubrows_v, [row_idx, uidx_v[sl] & (WIDE - 1)])
            ig = plsc.load_gather(
                ibrows_v, [row_idx, iidx_v[sl] & (WIDE - 1)])
            bsel_v[sl] = ug + ig

        pltpu.sync_copy(urows_v, urows_out.at[pl.ds(base, ROWS_PER_WORKER)])
        pltpu.sync_copy(irows_v, irows_out.at[pl.ds(base, ROWS_PER_WORKER)])
        pltpu.sync_copy(bsel_v, b_out.at[pl.ds(base, ROWS_PER_WORKER)])

    return k(user_ids, item_ids, uemb, iemb, ubias128, ibias128)


def _tc_dot_body(u_ref, i_ref, dot_ref):
    dot_ref[...] = jnp.sum(u_ref[...] * i_ref[...], axis=1).reshape(1, BATCH)


def _tc_dot(urows, irows):
    return pl.pallas_call(
        _tc_dot_body,
        out_shape=jax.ShapeDtypeStruct((1, BATCH), jnp.float32),
    )(urows, irows)


def _tc_bcast_body(dot_ref, b_ref, out_ref):
    out_ref[...] = dot_ref[...] + b_ref[...]


def _tc_broadcast(dot_row, bias):
    grid = (BATCH // ROWS_PER_BLOCK,)
    return pl.pallas_call(
        _tc_bcast_body,
        grid=grid,
        in_specs=[
            pl.BlockSpec((1, BATCH), lambda i: (0, 0)),
            pl.BlockSpec((ROWS_PER_BLOCK, 1), lambda i: (i, 0)),
        ],
        out_specs=pl.BlockSpec((ROWS_PER_BLOCK, BATCH), lambda i: (i, 0)),
        out_shape=jax.ShapeDtypeStruct((BATCH, BATCH), jnp.float32),
    )(dot_row, bias)


def kernel(user_ids, item_ids, user_emb_table, item_emb_table,
           user_bias_table, item_bias_table):
    def pad128(t):
        n = t.shape[0]
        pad = (-n) % WIDE
        flat = t.reshape(-1)
        if pad:
            flat = jnp.pad(flat, (0, pad))
        return flat.reshape(-1, WIDE)

    ubias128 = pad128(user_bias_table)
    ibias128 = pad128(item_bias_table)
    urows, irows, bias = _sc_gather(
        user_ids, item_ids, user_emb_table, item_emb_table,
        ubias128, ibias128)
    dot_row = _tc_dot(urows, irows)
    return _tc_broadcast(dot_row, bias.reshape(BATCH, 1))


# speedup vs baseline: 1.9640x; 1.0132x over previous
import functools

import jax
import jax.numpy as jnp
from jax import lax
from jax.experimental import pallas as pl
from jax.experimental.pallas import tpu as pltpu
from jax.experimental.pallas import tpu_sc as plsc

BATCH = 4096
DIM = 64
WIDE = 128
NUM_CORES = 2
NUM_SUBCORES = 16
NUM_WORKERS = NUM_CORES * NUM_SUBCORES
ROWS_PER_WORKER = BATCH // NUM_WORKERS
LANES = 16
ROWS_PER_BLOCK = 512


def _sc_gather(user_ids, item_ids, uemb, iemb, ubias128, ibias128):
    mesh = plsc.VectorSubcoreMesh(core_axis_name="c", subcore_axis_name="s")
    out_type = (
        jax.ShapeDtypeStruct((BATCH, DIM), jnp.float32),
        jax.ShapeDtypeStruct((BATCH, DIM), jnp.float32),
        jax.ShapeDtypeStruct((BATCH,), jnp.float32),
    )

    @functools.partial(
        pl.kernel,
        mesh=mesh,
        out_type=out_type,
        compiler_params=pltpu.CompilerParams(
            use_tc_tiling_on_sc=True, needs_layout_passes=False),
        scratch_types=[
            pltpu.VMEM((ROWS_PER_WORKER,), jnp.int32),
            pltpu.VMEM((ROWS_PER_WORKER,), jnp.int32),
            pltpu.VMEM((ROWS_PER_WORKER,), jnp.int32),
            pltpu.VMEM((ROWS_PER_WORKER,), jnp.int32),
            pltpu.VMEM((ROWS_PER_WORKER, DIM), jnp.float32),
            pltpu.VMEM((ROWS_PER_WORKER, DIM), jnp.float32),
            pltpu.VMEM((ROWS_PER_WORKER, WIDE), jnp.float32),
            pltpu.VMEM((ROWS_PER_WORKER, WIDE), jnp.float32),
            pltpu.VMEM((ROWS_PER_WORKER,), jnp.float32),
            pltpu.SemaphoreType.DMA,
            pltpu.SemaphoreType.DMA,
        ],
    )
    def k(uid_hbm, iid_hbm, uemb_hbm, iemb_hbm, ub_hbm, ib_hbm,
          urows_out, irows_out, b_out,
          uidx_v, iidx_v, ubrow_v, ibrow_v,
          urows_v, irows_v, ubrows_v, ibrows_v, bsel_v, sem, bsem):
        wid = lax.axis_index("s") * NUM_CORES + lax.axis_index("c")
        base = wid * ROWS_PER_WORKER
        pltpu.sync_copy(uid_hbm.at[pl.ds(base, ROWS_PER_WORKER)], uidx_v)
        pltpu.sync_copy(iid_hbm.at[pl.ds(base, ROWS_PER_WORKER)], iidx_v)

        @pl.loop(0, ROWS_PER_WORKER, step=LANES)
        def _(o):
            sl = pl.ds(o, LANES)
            ubrow_v[sl] = lax.shift_right_logical(uidx_v[sl], 7)
            ibrow_v[sl] = lax.shift_right_logical(iidx_v[sl], 7)

        cb1 = pltpu.async_copy(ub_hbm.at[ubrow_v], ubrows_v, bsem)
        cb2 = pltpu.async_copy(ib_hbm.at[ibrow_v], ibrows_v, bsem)

        @pl.loop(0, ROWS_PER_WORKER, step=LANES)
        def _(o):
            uv = uidx_v[pl.ds(o, LANES)]
            iv = iidx_v[pl.ds(o, LANES)]
            for jj in range(LANES):
                pltpu.async_copy(uemb_hbm.at[pl.ds(uv[jj], 1)],
                                 urows_v.at[pl.ds(o + jj, 1)], sem)
                pltpu.async_copy(iemb_hbm.at[pl.ds(iv[jj], 1)],
                                 irows_v.at[pl.ds(o + jj, 1)], sem)

        pltpu.make_async_copy(
            uemb_hbm.at[pl.ds(0, ROWS_PER_WORKER)], urows_v, sem).wait()
        pltpu.make_async_copy(
            iemb_hbm.at[pl.ds(0, ROWS_PER_WORKER)], irows_v, sem).wait()
        cb1.wait()
        cb2.wait()

        @pl.loop(0, ROWS_PER_WORKER, step=LANES)
        def _(o):
            sl = pl.ds(o, LANES)
            row_idx = o + lax.iota(jnp.int32, 16)
            ug = plsc.load_gather(
                ubrows_v, [row_idx, uidx_v[sl] & (WIDE - 1)])
            ig = plsc.load_gather(
                ibrows_v, [row_idx, iidx_v[sl] & (WIDE - 1)])
            bsel_v[sl] = ug + ig

        pltpu.sync_copy(urows_v, urows_out.at[pl.ds(base, ROWS_PER_WORKER)])
        pltpu.sync_copy(irows_v, irows_out.at[pl.ds(base, ROWS_PER_WORKER)])
        pltpu.sync_copy(bsel_v, b_out.at[pl.ds(base, ROWS_PER_WORKER)])

    return k(user_ids, item_ids, uemb, iemb, ubias128, ibias128)


def _tc_body(u_ref, i_ref, b_ref, out_ref, dot_ref):
    @pl.when(pl.program_id(0) == 0)
    def _():
        dot_ref[...] = jnp.sum(
            u_ref[...] * i_ref[...], axis=1).reshape(1, BATCH)
    out_ref[...] = dot_ref[...] + b_ref[...]


def _tc_broadcast(urows, irows, bias):
    grid = (BATCH // ROWS_PER_BLOCK,)
    return pl.pallas_call(
        _tc_body,
        grid=grid,
        in_specs=[
            pl.BlockSpec((BATCH, DIM), lambda i: (0, 0)),
            pl.BlockSpec((BATCH, DIM), lambda i: (0, 0)),
            pl.BlockSpec((ROWS_PER_BLOCK, 1), lambda i: (i, 0)),
        ],
        out_specs=pl.BlockSpec((ROWS_PER_BLOCK, BATCH), lambda i: (i, 0)),
        out_shape=jax.ShapeDtypeStruct((BATCH, BATCH), jnp.float32),
        scratch_shapes=[pltpu.VMEM((1, BATCH), jnp.float32)],
    )(urows, irows, bias)


def kernel(user_ids, item_ids, user_emb_table, item_emb_table,
           user_bias_table, item_bias_table):
    def pad128(t):
        n = t.shape[0]
        pad = (-n) % WIDE
        flat = t.reshape(-1)
        if pad:
            flat = jnp.pad(flat, (0, pad))
        return flat.reshape(-1, WIDE)

    ubias128 = pad128(user_bias_table)
    ibias128 = pad128(item_bias_table)
    urows, irows, bias = _sc_gather(
        user_ids, item_ids, user_emb_table, item_emb_table,
        ubias128, ibias128)
    return _tc_broadcast(urows, irows, bias.reshape(BATCH, 1))
